# baseline (speedup 1.0000x reference)
import functools

import jax
import jax.numpy as jnp
from jax import lax
from jax.experimental import pallas as pl
from jax.experimental.pallas import tpu as pltpu
from jax.experimental.pallas import tpu_sc as plsc

N = 10000
E = 320000
D = 128
NC = 2
NS = 16
NW = NC * NS
EPW = E // NW
CH = 80
NCHUNK = EPW // CH
CHA = 128
NFULL = EPW // CHA
TAIL = EPW - NFULL * CHA
NP = 10240
RPT = NP // NS

def _lrelu(x):
    return jnp.where(x >= 0, x, 0.01 * x)


@functools.cache
def _sc_kernels():
    mesh = plsc.VectorSubcoreMesh(
        core_axis_name="c", subcore_axis_name="s",
        num_cores=NC, num_subcores=NS)

    @functools.partial(
        pl.kernel,
        out_type=jax.ShapeDtypeStruct((NC, NP), jnp.float32),
        mesh=mesh,
        scratch_types=[
            pltpu.VMEM((NCHUNK, CH), jnp.int32),
            pltpu.VMEM((CH,), jnp.float32),
            pltpu.VMEM_SHARED((NP,), jnp.float32),
            pltpu.SemaphoreType.DMA,
        ],
        compiler_params=pltpu.CompilerParams(skip_device_barrier=True),
    )
    def deg_kernel(dst_hbm, ones_hbm, zhist_hbm, out_hbm,
                   dbuf_v, ones_v, hist_sh, sem):
        c = lax.axis_index("c")
        s = lax.axis_index("s")
        wid = c * NS + s
        pltpu.sync_copy(zhist_hbm, hist_sh.at[pl.ds(s * RPT, RPT)])
        pltpu.sync_copy(ones_hbm, ones_v)
        pltpu.sync_copy(dst_hbm.at[wid], dbuf_v)
        plsc.subcore_barrier()

        LAG = 8

        def body(i, carry):
            pltpu.async_copy(ones_v, hist_sh.at[dbuf_v.at[i]], sem, add=True)

            @pl.when(i >= LAG)
            def _():
                pltpu.make_async_copy(
                    ones_v, hist_sh.at[dbuf_v.at[i - LAG]], sem).wait()
            return carry

        lax.fori_loop(0, NCHUNK, body, 0)

        def drain(i, carry):
            pltpu.make_async_copy(
                ones_v, hist_sh.at[dbuf_v.at[NCHUNK - LAG + i]], sem).wait()
            return carry

        lax.fori_loop(0, LAG, drain, 0)
        plsc.subcore_barrier()
        pltpu.sync_copy(hist_sh.at[pl.ds(s * RPT, RPT)],
                        out_hbm.at[c].at[pl.ds(s * RPT, RPT)])

    @functools.partial(
        pl.kernel,
        out_type=jax.ShapeDtypeStruct((NC, NP, D), jnp.float32),
        mesh=mesh,
        scratch_types=[
            pltpu.VMEM((CHA,), jnp.int32),
            pltpu.VMEM((CHA,), jnp.int32),
            pltpu.VMEM((CHA,), jnp.int32),
            pltpu.VMEM((CHA,), jnp.int32),
            pltpu.VMEM((TAIL,), jnp.int32),
            pltpu.VMEM((TAIL,), jnp.int32),
            pltpu.VMEM((CHA, D), jnp.float32),
            pltpu.VMEM((CHA, D), jnp.float32),
            pltpu.VMEM((TAIL, D), jnp.float32),
            pltpu.VMEM_SHARED((NP, D), jnp.float32),
            pltpu.SemaphoreType.DMA,
            pltpu.SemaphoreType.DMA,
            pltpu.SemaphoreType.DMA,
            pltpu.SemaphoreType.DMA,
            pltpu.SemaphoreType.DMA,
            pltpu.SemaphoreType.DMA,
        ],
        compiler_params=pltpu.CompilerParams(skip_device_barrier=True),
    )
    def agg_kernel(y_hbm, src_hbm, dst_hbm, zrows_hbm, out_hbm,
                   sidx_a, didx_a, sidx_b, didx_b, sidx_t, didx_t,
                   rows_a, rows_b, rows_t, acc_sh,
                   sem_ga, sem_gb, sem_ia, sem_ib, sem_sa, sem_sb):
        c = lax.axis_index("c")
        s = lax.axis_index("s")
        wid = c * NS + s
        ebase = wid * EPW
        pltpu.sync_copy(zrows_hbm, acc_sh.at[pl.ds(s * RPT, RPT)])

        pltpu.sync_copy(src_hbm.at[pl.ds(ebase, CHA)], sidx_a)
        pltpu.sync_copy(dst_hbm.at[pl.ds(ebase, CHA)], didx_a)
        pltpu.sync_copy(src_hbm.at[pl.ds(ebase + CHA, CHA)], sidx_b)
        pltpu.sync_copy(dst_hbm.at[pl.ds(ebase + CHA, CHA)], didx_b)
        plsc.subcore_barrier()
        pltpu.async_copy(y_hbm.at[sidx_a], rows_a, sem_ga)
        pltpu.async_copy(y_hbm.at[sidx_b], rows_b, sem_gb)

        def body(i, carry):
            j = 2 * i
            pltpu.make_async_copy(y_hbm.at[sidx_a], rows_a, sem_ga).wait()
            pltpu.async_copy(rows_a, acc_sh.at[didx_a], sem_sa, add=True)
            pltpu.make_async_copy(y_hbm.at[sidx_b], rows_b, sem_gb).wait()
            pltpu.async_copy(rows_b, acc_sh.at[didx_b], sem_sb, add=True)

            pltpu.make_async_copy(rows_a, acc_sh.at[didx_a], sem_sa).wait()
            pltpu.async_copy(src_hbm.at[pl.ds(ebase + (j + 2) * CHA, CHA)],
                             sidx_a, sem_ia)
            pltpu.async_copy(dst_hbm.at[pl.ds(ebase + (j + 2) * CHA, CHA)],
                             didx_a, sem_ia)
            pltpu.make_async_copy(src_hbm.at[pl.ds(ebase, CHA)], sidx_a,
                                  sem_ia).wait()
            pltpu.make_async_copy(dst_hbm.at[pl.ds(ebase, CHA)], didx_a,
                                  sem_ia).wait()
            pltpu.async_copy(y_hbm.at[sidx_a], rows_a, sem_ga)

            pltpu.make_async_copy(rows_b, acc_sh.at[didx_b], sem_sb).wait()
            pltpu.async_copy(src_hbm.at[pl.ds(ebase + (j + 3) * CHA, CHA)],
                             sidx_b, sem_ib)
            pltpu.async_copy(dst_hbm.at[pl.ds(ebase + (j + 3) * CHA, CHA)],
                             didx_b, sem_ib)
            pltpu.make_async_copy(src_hbm.at[pl.ds(ebase, CHA)], sidx_b,
                                  sem_ib).wait()
            pltpu.make_async_copy(dst_hbm.at[pl.ds(ebase, CHA)], didx_b,
                                  sem_ib).wait()
            pltpu.async_copy(y_hbm.at[sidx_b], rows_b, sem_gb)
            return carry

        lax.fori_loop(0, (NFULL - 2) // 2, body, 0)
        pltpu.make_async_copy(y_hbm.at[sidx_a], rows_a, sem_ga).wait()
        pltpu.sync_copy(rows_a, acc_sh.at[didx_a], add=True)
        pltpu.make_async_copy(y_hbm.at[sidx_b], rows_b, sem_gb).wait()
        pltpu.sync_copy(rows_b, acc_sh.at[didx_b], add=True)
        pltpu.sync_copy(src_hbm.at[pl.ds(ebase + NFULL * CHA, TAIL)], sidx_t)
        pltpu.sync_copy(dst_hbm.at[pl.ds(ebase + NFULL * CHA, TAIL)], didx_t)
        pltpu.async_copy(y_hbm.at[sidx_t], rows_t, sem_ga).wait()
        pltpu.sync_copy(rows_t, acc_sh.at[didx_t], add=True)
        plsc.subcore_barrier()
        pltpu.sync_copy(acc_sh.at[pl.ds(s * RPT, RPT)],
                        out_hbm.at[c].at[pl.ds(s * RPT, RPT)])

    return deg_kernel, agg_kernel


def _enc1_body(prop, cat, tweet, des,
               W_num, b_num, W_bool, b_bool, W_tweet, b_tweet, W_des, b_des,
               W_lin1, b_lin1, W_gcn, xw_out):
    hn = _lrelu(jnp.dot(prop[...], W_num[...],
                        preferred_element_type=jnp.float32) + b_num[...])
    hb = _lrelu(cat[...] * W_bool[...][0][None, :] + b_bool[...])
    ht = _lrelu(jnp.dot(tweet[...], W_tweet[...],
                        preferred_element_type=jnp.float32) + b_tweet[...])
    hd = _lrelu(jnp.dot(des[...], W_des[...],
                        preferred_element_type=jnp.float32) + b_des[...])
    h = jnp.concatenate([hn, hb, ht, hd], axis=1)
    h = _lrelu(jnp.dot(h, W_lin1[...],
                       preferred_element_type=jnp.float32) + b_lin1[...])
    xw_out[...] = jnp.dot(h, W_gcn[...], preferred_element_type=jnp.float32)


def _tc_enc1(prop, cat, tweet, des,
             W_num, b_num, W_bool, b_bool, W_tweet, b_tweet, W_des, b_des,
             W_lin1, b_lin1, W_gcn):
    B = 1000
    row = lambda i: (i, 0)
    full = lambda i: (0, 0)
    vec = lambda i: (0,)
    return pl.pallas_call(
        _enc1_body,
        grid=(N // B,),
        in_specs=[
            pl.BlockSpec((B, 5), row),
            pl.BlockSpec((B, 1), row),
            pl.BlockSpec((B, 768), row),
            pl.BlockSpec((B, 768), row),
            pl.BlockSpec((5, 32), full),
            pl.BlockSpec((32,), vec),
            pl.BlockSpec((1, 32), full),
            pl.BlockSpec((32,), vec),
            pl.BlockSpec((768, 32), full),
            pl.BlockSpec((32,), vec),
            pl.BlockSpec((768, 32), full),
            pl.BlockSpec((32,), vec),
            pl.BlockSpec((128, 128), full),
            pl.BlockSpec((128,), vec),
            pl.BlockSpec((128, 128), full),
        ],
        out_specs=pl.BlockSpec((B, D), row),
        out_shape=jax.ShapeDtypeStruct((N, D), jnp.float32),
    )(prop, cat, tweet, des,
      W_num, b_num, W_bool, b_bool, W_tweet, b_tweet, W_des, b_des,
      W_lin1, b_lin1, W_gcn)


def _enc2_body(xw, deg, y_out, dinv_out):
    dinv = lax.rsqrt(deg[...])
    y_out[...] = xw[...] * dinv
    dinv_out[...] = dinv


def _tc_enc2(xw, deg):
    B = 1000
    row = lambda i: (i, 0)
    return pl.pallas_call(
        _enc2_body,
        grid=(N // B,),
        in_specs=[
            pl.BlockSpec((B, D), row),
            pl.BlockSpec((B, 1), row),
        ],
        out_specs=[
            pl.BlockSpec((B, D), row),
            pl.BlockSpec((B, 1), row),
        ],
        out_shape=[
            jax.ShapeDtypeStruct((N, D), jnp.float32),
            jax.ShapeDtypeStruct((N, 1), jnp.float32),
        ],
    )(xw, deg)


def _fin_body(acc0, acc1, y, dinv, b_gcn, W1, b1, W2, b2, out):
    g = (acc0[0] + acc1[0] + y[...]) * dinv[...] + b_gcn[...]
    t = _lrelu(jnp.dot(g, W1[...], preferred_element_type=jnp.float32)
               + b1[...])
    out[...] = jnp.dot(t, W2[...], preferred_element_type=jnp.float32) + b2[...]


def _tc_final(accp, y, dinv, b_gcn, W1, b1, W2, b2):
    B = 1000
    row = lambda i: (i, 0)
    full = lambda i: (0, 0)
    vec = lambda i: (0,)
    return pl.pallas_call(
        _fin_body,
        grid=(N // B,),
        in_specs=[
            pl.BlockSpec((1, B, D), lambda i: (0, i, 0)),
            pl.BlockSpec((1, B, D), lambda i: (1, i, 0)),
            pl.BlockSpec((B, D), row),
            pl.BlockSpec((B, 1), row),
            pl.BlockSpec((128,), vec),
            pl.BlockSpec((128, 64), full),
            pl.BlockSpec((64,), vec),
            pl.BlockSpec((64, 2), full),
            pl.BlockSpec((2,), vec),
        ],
        out_specs=pl.BlockSpec((B, 2), row),
        out_shape=jax.ShapeDtypeStruct((N, 2), jnp.float32),
    )(accp, accp, y, dinv, b_gcn, W1, b1, W2, b2)


def kernel(des_features, tweet_features, prop_features, cat_features,
           edge_index, edge_type, W_num, b_num, W_bool, b_bool,
           W_tweet, b_tweet, W_des, b_des, W_lin1, b_lin1, W_gcn, b_gcn,
           W_out1, b_out1, W_out2, b_out2):
    src = edge_index[0]
    dst = edge_index[1]
    dst3 = dst.reshape(NW, NCHUNK, CH)

    ones_vec = jnp.ones((CH,), jnp.float32)
    zhist = jnp.zeros((RPT,), jnp.float32)
    zrows = jnp.zeros((RPT, D), jnp.float32)

    deg_kernel, agg_kernel = _sc_kernels()
    hist = deg_kernel(dst3, ones_vec, zhist)
    deg = (hist[0, :N] + hist[1, :N] + 1.0)[:, None]

    xw = _tc_enc1(
        prop_features, cat_features, tweet_features, des_features,
        W_num, b_num, W_bool, b_bool, W_tweet, b_tweet, W_des, b_des,
        W_lin1, b_lin1, W_gcn)
    y, dinv = _tc_enc2(xw, deg)

    accp = agg_kernel(y, src, dst, zrows)

    return _tc_final(accp, y, dinv, b_gcn, W_out1, b_out1, W_out2, b_out2)


# speedup vs baseline: 1.0044x; 1.0044x over previous
import functools

import jax
import jax.numpy as jnp
from jax import lax
from jax.experimental import pallas as pl
from jax.experimental.pallas import tpu as pltpu
from jax.experimental.pallas import tpu_sc as plsc

N = 10000
E = 320000
D = 128
NC = 2
NS = 16
NW = NC * NS
EPW = E // NW
CH = 80
NCHUNK = EPW // CH
CHA = 128
NFULL = EPW // CHA
TAIL = EPW - NFULL * CHA
NP = 10240
RPT = NP // NS

def _lrelu(x):
    return jnp.where(x >= 0, x, 0.01 * x)


@functools.cache
def _sc_kernels():
    mesh = plsc.VectorSubcoreMesh(
        core_axis_name="c", subcore_axis_name="s",
        num_cores=NC, num_subcores=NS)

    @functools.partial(
        pl.kernel,
        out_type=jax.ShapeDtypeStruct((NC, NP), jnp.float32),
        mesh=mesh,
        scratch_types=[
            pltpu.VMEM((NCHUNK, CH), jnp.int32),
            pltpu.VMEM((CH,), jnp.float32),
            pltpu.VMEM_SHARED((NP,), jnp.float32),
            pltpu.SemaphoreType.DMA,
        ],
    )
    def deg_kernel(dst_hbm, ones_hbm, zhist_hbm, out_hbm,
                   dbuf_v, ones_v, hist_sh, sem):
        c = lax.axis_index("c")
        s = lax.axis_index("s")
        wid = c * NS + s
        pltpu.sync_copy(zhist_hbm, hist_sh.at[pl.ds(s * RPT, RPT)])
        pltpu.sync_copy(ones_hbm, ones_v)
        pltpu.sync_copy(dst_hbm.at[wid], dbuf_v)
        plsc.subcore_barrier()

        LAG = 8

        def body(i, carry):
            pltpu.async_copy(ones_v, hist_sh.at[dbuf_v.at[i]], sem, add=True)

            @pl.when(i >= LAG)
            def _():
                pltpu.make_async_copy(
                    ones_v, hist_sh.at[dbuf_v.at[i - LAG]], sem).wait()
            return carry

        lax.fori_loop(0, NCHUNK, body, 0)

        def drain(i, carry):
            pltpu.make_async_copy(
                ones_v, hist_sh.at[dbuf_v.at[NCHUNK - LAG + i]], sem).wait()
            return carry

        lax.fori_loop(0, LAG, drain, 0)
        plsc.subcore_barrier()
        pltpu.sync_copy(hist_sh.at[pl.ds(s * RPT, RPT)],
                        out_hbm.at[c].at[pl.ds(s * RPT, RPT)])

    @functools.partial(
        pl.kernel,
        out_type=jax.ShapeDtypeStruct((NC, NP, D), jnp.float32),
        mesh=mesh,
        scratch_types=[
            pltpu.VMEM((CHA,), jnp.int32),
            pltpu.VMEM((CHA,), jnp.int32),
            pltpu.VMEM((CHA,), jnp.int32),
            pltpu.VMEM((CHA,), jnp.int32),
            pltpu.VMEM((TAIL,), jnp.int32),
            pltpu.VMEM((TAIL,), jnp.int32),
            pltpu.VMEM((CHA, D), jnp.float32),
            pltpu.VMEM((CHA, D), jnp.float32),
            pltpu.VMEM((TAIL, D), jnp.float32),
            pltpu.VMEM_SHARED((NP, D), jnp.float32),
            pltpu.SemaphoreType.DMA,
            pltpu.SemaphoreType.DMA,
            pltpu.SemaphoreType.DMA,
            pltpu.SemaphoreType.DMA,
            pltpu.SemaphoreType.DMA,
            pltpu.SemaphoreType.DMA,
        ],
    )
    def agg_kernel(y_hbm, src_hbm, dst_hbm, zrows_hbm, out_hbm,
                   sidx_a, didx_a, sidx_b, didx_b, sidx_t, didx_t,
                   rows_a, rows_b, rows_t, acc_sh,
                   sem_ga, sem_gb, sem_ia, sem_ib, sem_sa, sem_sb):
        c = lax.axis_index("c")
        s = lax.axis_index("s")
        wid = c * NS + s
        ebase = wid * EPW
        pltpu.sync_copy(zrows_hbm, acc_sh.at[pl.ds(s * RPT, RPT)])

        pltpu.sync_copy(src_hbm.at[pl.ds(ebase, CHA)], sidx_a)
        pltpu.sync_copy(dst_hbm.at[pl.ds(ebase, CHA)], didx_a)
        pltpu.sync_copy(src_hbm.at[pl.ds(ebase + CHA, CHA)], sidx_b)
        pltpu.sync_copy(dst_hbm.at[pl.ds(ebase + CHA, CHA)], didx_b)
        plsc.subcore_barrier()
        pltpu.async_copy(y_hbm.at[sidx_a], rows_a, sem_ga)
        pltpu.async_copy(y_hbm.at[sidx_b], rows_b, sem_gb)

        def body(i, carry):
            j = 2 * i
            pltpu.make_async_copy(y_hbm.at[sidx_a], rows_a, sem_ga).wait()
            pltpu.async_copy(rows_a, acc_sh.at[didx_a], sem_sa, add=True)
            pltpu.make_async_copy(y_hbm.at[sidx_b], rows_b, sem_gb).wait()
            pltpu.async_copy(rows_b, acc_sh.at[didx_b], sem_sb, add=True)

            pltpu.make_async_copy(rows_a, acc_sh.at[didx_a], sem_sa).wait()
            pltpu.async_copy(src_hbm.at[pl.ds(ebase + (j + 2) * CHA, CHA)],
                             sidx_a, sem_ia)
            pltpu.async_copy(dst_hbm.at[pl.ds(ebase + (j + 2) * CHA, CHA)],
                             didx_a, sem_ia)
            pltpu.make_async_copy(src_hbm.at[pl.ds(ebase, CHA)], sidx_a,
                                  sem_ia).wait()
            pltpu.make_async_copy(dst_hbm.at[pl.ds(ebase, CHA)], didx_a,
                                  sem_ia).wait()
            pltpu.async_copy(y_hbm.at[sidx_a], rows_a, sem_ga)

            pltpu.make_async_copy(rows_b, acc_sh.at[didx_b], sem_sb).wait()
            pltpu.async_copy(src_hbm.at[pl.ds(ebase + (j + 3) * CHA, CHA)],
                             sidx_b, sem_ib)
            pltpu.async_copy(dst_hbm.at[pl.ds(ebase + (j + 3) * CHA, CHA)],
                             didx_b, sem_ib)
            pltpu.make_async_copy(src_hbm.at[pl.ds(ebase, CHA)], sidx_b,
                                  sem_ib).wait()
            pltpu.make_async_copy(dst_hbm.at[pl.ds(ebase, CHA)], didx_b,
                                  sem_ib).wait()
            pltpu.async_copy(y_hbm.at[sidx_b], rows_b, sem_gb)
            return carry

        lax.fori_loop(0, (NFULL - 2) // 2, body, 0)
        pltpu.make_async_copy(y_hbm.at[sidx_a], rows_a, sem_ga).wait()
        pltpu.sync_copy(rows_a, acc_sh.at[didx_a], add=True)
        pltpu.make_async_copy(y_hbm.at[sidx_b], rows_b, sem_gb).wait()
        pltpu.sync_copy(rows_b, acc_sh.at[didx_b], add=True)
        pltpu.sync_copy(src_hbm.at[pl.ds(ebase + NFULL * CHA, TAIL)], sidx_t)
        pltpu.sync_copy(dst_hbm.at[pl.ds(ebase + NFULL * CHA, TAIL)], didx_t)
        pltpu.async_copy(y_hbm.at[sidx_t], rows_t, sem_ga).wait()
        pltpu.sync_copy(rows_t, acc_sh.at[didx_t], add=True)
        plsc.subcore_barrier()
        pltpu.sync_copy(acc_sh.at[pl.ds(s * RPT, RPT)],
                        out_hbm.at[c].at[pl.ds(s * RPT, RPT)])

    return deg_kernel, agg_kernel


def _enc1_body(prop, cat, tweet, des,
               W_num, b_num, W_bool, b_bool, W_tweet, b_tweet, W_des, b_des,
               W_lin1, b_lin1, W_gcn, xw_out):
    hn = _lrelu(jnp.dot(prop[...], W_num[...],
                        preferred_element_type=jnp.float32) + b_num[...])
    hb = _lrelu(cat[...] * W_bool[...][0][None, :] + b_bool[...])
    ht = _lrelu(jnp.dot(tweet[...], W_tweet[...],
                        preferred_element_type=jnp.float32) + b_tweet[...])
    hd = _lrelu(jnp.dot(des[...], W_des[...],
                        preferred_element_type=jnp.float32) + b_des[...])
    h = jnp.concatenate([hn, hb, ht, hd], axis=1)
    h = _lrelu(jnp.dot(h, W_lin1[...],
                       preferred_element_type=jnp.float32) + b_lin1[...])
    xw_out[...] = jnp.dot(h, W_gcn[...], preferred_element_type=jnp.float32)


def _tc_enc1(prop, cat, tweet, des,
             W_num, b_num, W_bool, b_bool, W_tweet, b_tweet, W_des, b_des,
             W_lin1, b_lin1, W_gcn):
    B = 2000
    row = lambda i: (i, 0)
    full = lambda i: (0, 0)
    vec = lambda i: (0,)
    return pl.pallas_call(
        _enc1_body,
        grid=(N // B,),
        in_specs=[
            pl.BlockSpec((B, 5), row),
            pl.BlockSpec((B, 1), row),
            pl.BlockSpec((B, 768), row),
            pl.BlockSpec((B, 768), row),
            pl.BlockSpec((5, 32), full),
            pl.BlockSpec((32,), vec),
            pl.BlockSpec((1, 32), full),
            pl.BlockSpec((32,), vec),
            pl.BlockSpec((768, 32), full),
            pl.BlockSpec((32,), vec),
            pl.BlockSpec((768, 32), full),
            pl.BlockSpec((32,), vec),
            pl.BlockSpec((128, 128), full),
            pl.BlockSpec((128,), vec),
            pl.BlockSpec((128, 128), full),
        ],
        out_specs=pl.BlockSpec((B, D), row),
        out_shape=jax.ShapeDtypeStruct((N, D), jnp.float32),
    )(prop, cat, tweet, des,
      W_num, b_num, W_bool, b_bool, W_tweet, b_tweet, W_des, b_des,
      W_lin1, b_lin1, W_gcn)


def _enc2_body(xw, deg, y_out, dinv_out):
    dinv = lax.rsqrt(deg[...])
    y_out[...] = xw[...] * dinv
    dinv_out[...] = dinv


def _tc_enc2(xw, deg):
    B = 1000
    row = lambda i: (i, 0)
    return pl.pallas_call(
        _enc2_body,
        grid=(N // B,),
        in_specs=[
            pl.BlockSpec((B, D), row),
            pl.BlockSpec((B, 1), row),
        ],
        out_specs=[
            pl.BlockSpec((B, D), row),
            pl.BlockSpec((B, 1), row),
        ],
        out_shape=[
            jax.ShapeDtypeStruct((N, D), jnp.float32),
            jax.ShapeDtypeStruct((N, 1), jnp.float32),
        ],
    )(xw, deg)


def _fin_body(acc0, acc1, y, dinv, b_gcn, W1, b1, W2, b2, out):
    g = (acc0[0] + acc1[0] + y[...]) * dinv[...] + b_gcn[...]
    t = _lrelu(jnp.dot(g, W1[...], preferred_element_type=jnp.float32)
               + b1[...])
    out[...] = jnp.dot(t, W2[...], preferred_element_type=jnp.float32) + b2[...]


def _tc_final(accp, y, dinv, b_gcn, W1, b1, W2, b2):
    B = 1000
    row = lambda i: (i, 0)
    full = lambda i: (0, 0)
    vec = lambda i: (0,)
    return pl.pallas_call(
        _fin_body,
        grid=(N // B,),
        in_specs=[
            pl.BlockSpec((1, B, D), lambda i: (0, i, 0)),
            pl.BlockSpec((1, B, D), lambda i: (1, i, 0)),
            pl.BlockSpec((B, D), row),
            pl.BlockSpec((B, 1), row),
            pl.BlockSpec((128,), vec),
            pl.BlockSpec((128, 64), full),
            pl.BlockSpec((64,), vec),
            pl.BlockSpec((64, 2), full),
            pl.BlockSpec((2,), vec),
        ],
        out_specs=pl.BlockSpec((B, 2), row),
        out_shape=jax.ShapeDtypeStruct((N, 2), jnp.float32),
    )(accp, accp, y, dinv, b_gcn, W1, b1, W2, b2)


def kernel(des_features, tweet_features, prop_features, cat_features,
           edge_index, edge_type, W_num, b_num, W_bool, b_bool,
           W_tweet, b_tweet, W_des, b_des, W_lin1, b_lin1, W_gcn, b_gcn,
           W_out1, b_out1, W_out2, b_out2):
    src = edge_index[0]
    dst = edge_index[1]
    dst3 = dst.reshape(NW, NCHUNK, CH)

    ones_vec = jnp.ones((CH,), jnp.float32)
    zhist = jnp.zeros((RPT,), jnp.float32)
    zrows = jnp.zeros((RPT, D), jnp.float32)

    deg_kernel, agg_kernel = _sc_kernels()
    hist = deg_kernel(dst3, ones_vec, zhist)
    deg = (hist[0, :N] + hist[1, :N] + 1.0)[:, None]

    xw = _tc_enc1(
        prop_features, cat_features, tweet_features, des_features,
        W_num, b_num, W_bool, b_bool, W_tweet, b_tweet, W_des, b_des,
        W_lin1, b_lin1, W_gcn)
    y, dinv = _tc_enc2(xw, deg)

    accp = agg_kernel(y, src, dst, zrows)

    return _tc_final(accp, y, dinv, b_gcn, W_out1, b_out1, W_out2, b_out2)
